# Initial kernel scaffold; baseline (speedup 1.0000x reference)
import jax
import jax.numpy as jnp
from jax.experimental import pallas as pl


def kernel(input, conv_w, conv_b, ln_g, ln_b, gate_w, w1, b1, w2, b2, ls):
    raise NotImplementedError("write your pallas kernel here")


# speedup vs baseline: 4.1399x; 4.1399x over previous
import functools

import jax
import jax.numpy as jnp
from jax.experimental import pallas as pl
from jax.experimental.pallas import tpu as pltpu

DIM = 96
E = 8
H = 224
W = 224
STRIP = 8
GRID = H // STRIP
WPAD = 232
HPAD = 240
EPS = 1e-06


def _moecn_kernel(xa_ref, xb_ref, k_ref, cb_ref, lng_ref, lnb_ref,
                  gwt_ref, w1t_ref, b1_ref, w2t_ref, b2_ref, out_ref):
    rows = jnp.concatenate([xa_ref[...], xb_ref[...]], axis=0)

    acc = jnp.zeros((STRIP, W, DIM), jnp.float32)
    for dw in range(7):
        xs = rows[:, dw:dw + W, :]
        for dh in range(7):
            tap = k_ref[dh * 7 + dw][None, None, :]
            acc = acc + xs[dh:dh + STRIP] * tap
    xc = acc + cb_ref[0][None, None, :]

    x2 = xc.reshape(STRIP * W, DIM)
    mu = jnp.mean(x2, axis=-1, keepdims=True)
    var = jnp.mean(x2 * x2, axis=-1, keepdims=True) - mu * mu
    xln = (x2 - mu) * jax.lax.rsqrt(var + EPS) * lng_ref[0][None, :] \
        + lnb_ref[0][None, :]

    logits = jnp.dot(xln, gwt_ref[...], preferred_element_type=jnp.float32)
    eid = jnp.argmax(logits, axis=-1)

    y = jnp.zeros((STRIP * W, DIM), jnp.float32)
    for e in range(E):
        hpre = jnp.dot(xln, w1t_ref[e], preferred_element_type=jnp.float32) \
            + b1_ref[e][None, :]
        hact = 0.5 * hpre * (1.0 + jax.lax.erf(hpre * 0.7071067811865476))
        ye = jnp.dot(hact, w2t_ref[e], preferred_element_type=jnp.float32) \
            + b2_ref[e][None, :]
        mask = (eid == e).astype(jnp.float32)[:, None]
        y = y + mask * ye
    out_ref[...] = y.reshape(STRIP, W, DIM)


@functools.partial(jax.jit, static_argnums=())
def kernel(input, conv_w, conv_b, ln_g, ln_b, gate_w, w1, b1, w2, b2, ls):
    x_nhwc = jnp.transpose(input[0], (1, 2, 0))
    x_pad = jnp.pad(x_nhwc, ((3, HPAD - H - 3), (3, WPAD - W - 3), (0, 0)))

    k = jnp.transpose(conv_w[:, 0, :, :], (1, 2, 0)).reshape(49, DIM)
    k = jnp.pad(k, ((0, 7), (0, 0)))

    spec_row = lambda off: pl.BlockSpec(
        (STRIP, WPAD, DIM), lambda i: (i + off, 0, 0))
    full = lambda a: pl.BlockSpec(a.shape, lambda i: (0,) * a.ndim)

    kb = k
    cb = conv_b.reshape(1, DIM)
    lng = ln_g.reshape(1, DIM)
    lnb = ln_b.reshape(1, DIM)
    gwt = gate_w.T
    w1t = jnp.transpose(w1, (0, 2, 1))
    w2t = jnp.transpose(w2, (0, 2, 1))

    y = pl.pallas_call(
        _moecn_kernel,
        grid=(GRID,),
        in_specs=[
            spec_row(0),
            spec_row(1),
            full(kb), full(cb), full(lng), full(lnb),
            full(gwt), full(w1t), full(b1), full(w2t), full(b2),
        ],
        out_specs=pl.BlockSpec((STRIP, W, DIM), lambda i: (i, 0, 0)),
        out_shape=jax.ShapeDtypeStruct((H, W, DIM), jnp.float32),
    )(x_pad, x_pad, kb, cb, lng, lnb, gwt, w1t, b1, w2t, b2)

    return input + ls[None] * jnp.transpose(y, (2, 0, 1))[None]


# speedup vs baseline: 6.0136x; 1.4526x over previous
import jax
import jax.numpy as jnp
from jax.experimental import pallas as pl

DIM = 96
E = 8
H = 224
W = 224
WP = 256
HP = 232
TB = 2048
GRID = HP * WP // TB
HALO = 3 * WP + 3
EPS = 1e-06


def _moecn_kernel(xp_ref, xc_ref, xn_ref, k_ref, cb_ref, lng_ref, lnb_ref,
                  gw_ref, w1s_ref, b1_ref, w2s_ref, b2_ref, out_ref):
    lo = TB - HALO
    acc = jnp.zeros((DIM, TB), jnp.float32)
    for dw in range(7):
        xw = jnp.concatenate(
            [xp_ref[:, lo + dw:], xc_ref[...], xn_ref[:, :HALO - 6 + dw]],
            axis=1)
        for dh in range(7):
            tap = k_ref[:, dh * 7 + dw:dh * 7 + dw + 1]
            acc = acc + xw[:, dh * WP:dh * WP + TB] * tap
    xc = acc + cb_ref[...]

    mu = jnp.mean(xc, axis=0, keepdims=True)
    var = jnp.mean(xc * xc, axis=0, keepdims=True) - mu * mu
    xln = (xc - mu) * jax.lax.rsqrt(var + EPS) * lng_ref[...] + lnb_ref[...]

    logits = jnp.dot(gw_ref[...], xln, preferred_element_type=jnp.float32)
    mx = jnp.max(logits, axis=0, keepdims=True)
    taken = jnp.zeros((1, TB), jnp.bool_)
    masks = []
    for e in range(E):
        hit = (logits[e:e + 1, :] == mx) & (~taken)
        taken = taken | hit
        masks.append(hit.astype(jnp.float32))

    xb = xln.astype(jnp.bfloat16)
    hall = jnp.dot(w1s_ref[...], xb, preferred_element_type=jnp.float32)
    hsel = jnp.zeros((DIM, TB), jnp.float32)
    for e in range(E):
        hsel = hsel + masks[e] * (hall[DIM * e:DIM * (e + 1), :]
                                  + b1_ref[:, e:e + 1])
    g = 0.5 * hsel * (1.0 + jax.lax.erf(hsel * 0.7071067811865476))

    gb = g.astype(jnp.bfloat16)
    gs = jnp.concatenate(
        [masks[e].astype(jnp.bfloat16) * gb for e in range(E)], axis=0)
    y = jnp.dot(w2s_ref[...], gs, preferred_element_type=jnp.float32)
    for e in range(E):
        y = y + masks[e] * b2_ref[:, e:e + 1]
    out_ref[...] = y


def kernel(input, conv_w, conv_b, ln_g, ln_b, gate_w, w1, b1, w2, b2, ls):
    x3 = jnp.pad(input[0], ((0, 0), (3, HP - H - 3), (3, WP - W - 3)))
    x2 = x3.reshape(DIM, HP * WP)

    k = conv_w[:, 0, :, :].reshape(DIM, 49)
    k = jnp.pad(k, ((0, 0), (0, 7)))
    cb = conv_b.reshape(DIM, 1)
    lng = ln_g.reshape(DIM, 1)
    lnb = ln_b.reshape(DIM, 1)
    w1s = w1.reshape(E * DIM, DIM).astype(jnp.bfloat16)
    w2s = jnp.transpose(w2, (1, 0, 2)).reshape(DIM, E * DIM) \
        .astype(jnp.bfloat16)
    b1t = b1.T
    b2t = b2.T

    blk = lambda off: pl.BlockSpec(
        (DIM, TB),
        lambda i: (0, jnp.clip(i + off, 0, GRID - 1)))
    full = lambda a: pl.BlockSpec(a.shape, lambda i: (0,) * a.ndim)

    y = pl.pallas_call(
        _moecn_kernel,
        grid=(GRID,),
        in_specs=[
            blk(-1), blk(0), blk(1),
            full(k), full(cb), full(lng), full(lnb),
            full(gate_w), full(w1s), full(b1t), full(w2s), full(b2t),
        ],
        out_specs=pl.BlockSpec((DIM, TB), lambda i: (0, i)),
        out_shape=jax.ShapeDtypeStruct((DIM, HP * WP), jnp.float32),
    )(x2, x2, x2, k, cb, lng, lnb, gate_w, w1s, b1t, w2s, b2t)

    yc = y.reshape(DIM, HP, WP)[:, 3:3 + H, 3:3 + W]
    return input + ls[None] * yc[None]


# speedup vs baseline: 7.5634x; 1.2577x over previous
import jax
import jax.numpy as jnp
from jax.experimental import pallas as pl

DIM = 96
E = 8
H = 224
W = 224
WP = 256
HP = 232
TB = 2048
GRID = HP * WP // TB
HALO = 3 * WP + 3
EPS = 1e-06


def _moecn_kernel(xp_ref, xc_ref, xn_ref, k_ref, cb_ref, lng_ref, lnb_ref,
                  gw_ref, w1s_ref, b1_ref, w2s_ref, b2_ref, out_ref):
    lo = TB - HALO
    acc = jnp.zeros((DIM, TB), jnp.bfloat16)
    for dw in range(7):
        xw = jnp.concatenate(
            [xp_ref[:, lo + dw:], xc_ref[...], xn_ref[:, :HALO - 6 + dw]],
            axis=1)
        for dh in range(7):
            tap = k_ref[:, dh * 7 + dw:dh * 7 + dw + 1]
            acc = acc + xw[:, dh * WP:dh * WP + TB] * tap
    xc = acc.astype(jnp.float32) + cb_ref[...]

    mu = jnp.mean(xc, axis=0, keepdims=True)
    var = jnp.mean(xc * xc, axis=0, keepdims=True) - mu * mu
    xln = (xc - mu) * jax.lax.rsqrt(var + EPS) * lng_ref[...] + lnb_ref[...]

    logits = jnp.dot(gw_ref[...], xln, preferred_element_type=jnp.float32)
    mx = jnp.max(logits, axis=0, keepdims=True)
    taken = jnp.zeros((1, TB), jnp.bool_)
    masks = []
    for e in range(E):
        hit = (logits[e:e + 1, :] == mx) & (~taken)
        taken = taken | hit
        masks.append(hit.astype(jnp.float32))

    xb = xln.astype(jnp.bfloat16)
    hall = jnp.dot(w1s_ref[...], xb, preferred_element_type=jnp.float32)
    hsel = jnp.zeros((DIM, TB), jnp.float32)
    for e in range(E):
        hsel = hsel + masks[e] * (hall[DIM * e:DIM * (e + 1), :]
                                  + b1_ref[:, e:e + 1])
    g = 0.5 * hsel * (1.0 + jax.lax.erf(hsel * 0.7071067811865476))

    gb = g.astype(jnp.bfloat16)
    gs = jnp.concatenate(
        [masks[e].astype(jnp.bfloat16) * gb for e in range(E)], axis=0)
    y = jnp.dot(w2s_ref[...], gs, preferred_element_type=jnp.float32)
    for e in range(E):
        y = y + masks[e] * b2_ref[:, e:e + 1]
    out_ref[...] = y


def kernel(input, conv_w, conv_b, ln_g, ln_b, gate_w, w1, b1, w2, b2, ls):
    x3 = jnp.pad(input[0], ((0, 0), (3, HP - H - 3), (3, WP - W - 3)))
    x2 = x3.reshape(DIM, HP * WP).astype(jnp.bfloat16)

    k = conv_w[:, 0, :, :].reshape(DIM, 49)
    k = jnp.pad(k, ((0, 0), (0, 7))).astype(jnp.bfloat16)
    cb = conv_b.reshape(DIM, 1)
    lng = ln_g.reshape(DIM, 1)
    lnb = ln_b.reshape(DIM, 1)
    w1s = w1.reshape(E * DIM, DIM).astype(jnp.bfloat16)
    w2s = jnp.transpose(w2, (1, 0, 2)).reshape(DIM, E * DIM) \
        .astype(jnp.bfloat16)
    b1t = b1.T
    b2t = b2.T

    blk = lambda off: pl.BlockSpec(
        (DIM, TB),
        lambda i: (0, jnp.clip(i + off, 0, GRID - 1)))
    full = lambda a: pl.BlockSpec(a.shape, lambda i: (0,) * a.ndim)

    y = pl.pallas_call(
        _moecn_kernel,
        grid=(GRID,),
        in_specs=[
            blk(-1), blk(0), blk(1),
            full(k), full(cb), full(lng), full(lnb),
            full(gate_w), full(w1s), full(b1t), full(w2s), full(b2t),
        ],
        out_specs=pl.BlockSpec((DIM, TB), lambda i: (0, i)),
        out_shape=jax.ShapeDtypeStruct((DIM, HP * WP), jnp.float32),
    )(x2, x2, x2, k, cb, lng, lnb, gate_w, w1s, b1t, w2s, b2t)

    yc = y.reshape(DIM, HP, WP)[:, 3:3 + H, 3:3 + W]
    return input + ls[None] * yc[None]


# speedup vs baseline: 8.1217x; 1.0738x over previous
import jax
import jax.numpy as jnp
from jax.experimental import pallas as pl

DIM = 96
E = 8
H = 224
W = 224
WP = 256
HP = 232
TB = 2048
GRID = HP * WP // TB
HALO = 3 * WP + 3
EPS = 1e-06


def _moecn_kernel(xp_ref, xc_ref, xn_ref, k_ref, cb_ref, lng_ref, lnb_ref,
                  gw_ref, w1s_ref, w2s_ref, out_ref):
    lo = TB - HALO
    acc = jnp.zeros((DIM, TB), jnp.bfloat16)
    for dw in range(7):
        xw = jnp.concatenate(
            [xp_ref[:, lo + dw:], xc_ref[...], xn_ref[:, :HALO - 6 + dw]],
            axis=1)
        for dh in range(7):
            tap = k_ref[:, dh * 7 + dw:dh * 7 + dw + 1]
            acc = acc + xw[:, dh * WP:dh * WP + TB] * tap
    xc = acc.astype(jnp.float32) + cb_ref[...]

    mu = jnp.mean(xc, axis=0, keepdims=True)
    var = jnp.mean(xc * xc, axis=0, keepdims=True) - mu * mu
    xln = (xc - mu) * jax.lax.rsqrt(var + EPS) * lng_ref[...] + lnb_ref[...]

    logits = jnp.dot(gw_ref[...], xln, preferred_element_type=jnp.float32)
    mx = jnp.max(logits, axis=0, keepdims=True)
    taken = jnp.zeros((1, TB), jnp.bool_)
    masks = []
    for e in range(E):
        hit = (logits[e:e + 1, :] == mx) & (~taken)
        taken = taken | hit
        masks.append(hit.astype(jnp.bfloat16))
    zrow = jnp.zeros((E, TB), jnp.bfloat16)

    xb = xln.astype(jnp.bfloat16)
    xs = jnp.concatenate([masks[e] * xb for e in range(E)]
                         + masks + [zrow], axis=0)
    hsel = jnp.dot(w1s_ref[...], xs, preferred_element_type=jnp.float32)
    g = 0.5 * hsel * (1.0 + jax.lax.erf(hsel * 0.7071067811865476))

    gb = g.astype(jnp.bfloat16)
    gs = jnp.concatenate([masks[e] * gb for e in range(E)]
                         + masks + [zrow], axis=0)
    y = jnp.dot(w2s_ref[...], gs, preferred_element_type=jnp.float32)
    out_ref[...] = y


def kernel(input, conv_w, conv_b, ln_g, ln_b, gate_w, w1, b1, w2, b2, ls):
    x3 = jnp.pad(input[0], ((0, 0), (3, HP - H - 3), (3, WP - W - 3)))
    x2 = x3.reshape(DIM, HP * WP).astype(jnp.bfloat16)

    k = conv_w[:, 0, :, :].reshape(DIM, 49)
    k = jnp.pad(k, ((0, 0), (0, 7))).astype(jnp.bfloat16)
    cb = conv_b.reshape(DIM, 1)
    lng = ln_g.reshape(DIM, 1)
    lnb = ln_b.reshape(DIM, 1)
    zcol = jnp.zeros((DIM, E), jnp.bfloat16)
    w1s = jnp.concatenate(
        [jnp.transpose(w1, (1, 0, 2)).reshape(DIM, E * DIM), b1.T, zcol],
        axis=1).astype(jnp.bfloat16)
    w2s = jnp.concatenate(
        [jnp.transpose(w2, (1, 0, 2)).reshape(DIM, E * DIM), b2.T, zcol],
        axis=1).astype(jnp.bfloat16)

    blk = lambda off: pl.BlockSpec(
        (DIM, TB),
        lambda i: (0, jnp.clip(i + off, 0, GRID - 1)))
    full = lambda a: pl.BlockSpec(a.shape, lambda i: (0,) * a.ndim)

    y = pl.pallas_call(
        _moecn_kernel,
        grid=(GRID,),
        in_specs=[
            blk(-1), blk(0), blk(1),
            full(k), full(cb), full(lng), full(lnb),
            full(gate_w), full(w1s), full(w2s),
        ],
        out_specs=pl.BlockSpec((DIM, TB), lambda i: (0, i)),
        out_shape=jax.ShapeDtypeStruct((DIM, HP * WP), jnp.float32),
    )(x2, x2, x2, k, cb, lng, lnb, gate_w, w1s, w2s)

    yc = y.reshape(DIM, HP, WP)[:, 3:3 + H, 3:3 + W]
    return input + ls[None] * yc[None]


# speedup vs baseline: 10.0435x; 1.2366x over previous
import jax
import jax.numpy as jnp
from jax.experimental import pallas as pl
from jax.experimental.pallas import tpu as pltpu

DIM = 96
E = 8
H = 224
W = 224
WP = 256
HP = 232
TB = 2048
GRID = HP * WP // TB
HALO = 3 * WP + 3
EPS = 1e-06


def _moecn_kernel(xp_ref, xc_ref, xn_ref, k_ref, cb_ref, lng_ref, lnb_ref,
                  gw_ref, w1s_ref, w2s_ref, out_ref, xln_s):
    xln = xln_s[...]
    logits = jnp.dot(gw_ref[...], xln, preferred_element_type=jnp.float32)
    mx = jnp.max(logits, axis=0, keepdims=True)
    taken = jnp.zeros((1, TB), jnp.bool_)
    masks = []
    for e in range(E):
        hit = (logits[e:e + 1, :] == mx) & (~taken)
        taken = taken | hit
        masks.append(hit.astype(jnp.bfloat16))
    zrow = jnp.zeros((E, TB), jnp.bfloat16)

    xb = xln.astype(jnp.bfloat16)
    xs = jnp.concatenate([masks[e] * xb for e in range(E)]
                         + masks + [zrow], axis=0)
    hsel = jnp.dot(w1s_ref[...], xs, preferred_element_type=jnp.float32)
    g = 0.5 * hsel * (1.0 + jax.lax.erf(hsel * 0.7071067811865476))
    gb = g.astype(jnp.bfloat16)
    gs = jnp.concatenate([masks[e] * gb for e in range(E)]
                         + masks + [zrow], axis=0)
    y = jnp.dot(w2s_ref[...], gs, preferred_element_type=jnp.float32)
    out_ref[...] = y.astype(jnp.bfloat16)

    lo = TB - HALO
    parts = []
    for dw in range(7):
        xw = jnp.concatenate(
            [xp_ref[:, lo + dw:], xc_ref[...], xn_ref[:, :HALO - 6 + dw]],
            axis=1)
        p = None
        for dh in range(7):
            tap = k_ref[:, dh * 7 + dw:dh * 7 + dw + 1]
            t = xw[:, dh * WP:dh * WP + TB] * tap
            p = t if p is None else p + t
        parts.append(p)
    acc = ((parts[0] + parts[1]) + (parts[2] + parts[3])) \
        + ((parts[4] + parts[5]) + parts[6])
    xc = acc.astype(jnp.float32) + cb_ref[...]

    mu = jnp.mean(xc, axis=0, keepdims=True)
    var = jnp.mean(xc * xc, axis=0, keepdims=True) - mu * mu
    xln_s[...] = ((xc - mu) * jax.lax.rsqrt(var + EPS) * lng_ref[...]
                  + lnb_ref[...])


def kernel(input, conv_w, conv_b, ln_g, ln_b, gate_w, w1, b1, w2, b2, ls):
    x3 = jnp.pad(input[0], ((0, 0), (3, HP - H - 3), (3, WP - W - 3)))
    x2 = x3.reshape(DIM, HP * WP).astype(jnp.bfloat16)

    k = conv_w[:, 0, :, :].reshape(DIM, 49)
    k = jnp.pad(k, ((0, 0), (0, 7))).astype(jnp.bfloat16)
    cb = conv_b.reshape(DIM, 1)
    lng = ln_g.reshape(DIM, 1)
    lnb = ln_b.reshape(DIM, 1)
    zcol = jnp.zeros((DIM, E), jnp.float32)
    w1s = jnp.concatenate(
        [jnp.transpose(w1, (1, 0, 2)).reshape(DIM, E * DIM), b1.T, zcol],
        axis=1).astype(jnp.bfloat16)
    w2s = jnp.concatenate(
        [jnp.transpose(w2, (1, 0, 2)).reshape(DIM, E * DIM), b2.T, zcol],
        axis=1).astype(jnp.bfloat16)

    blk = lambda f: pl.BlockSpec((DIM, TB), lambda i: (0, f(i)))
    full = lambda a: pl.BlockSpec(a.shape, lambda i: (0,) * a.ndim)

    y = pl.pallas_call(
        _moecn_kernel,
        grid=(GRID + 1,),
        in_specs=[
            blk(lambda i: jnp.clip(i - 1, 0, GRID - 1)),
            blk(lambda i: jnp.minimum(i, GRID - 1)),
            blk(lambda i: jnp.minimum(i + 1, GRID - 1)),
            full(k), full(cb), full(lng), full(lnb),
            full(gate_w), full(w1s), full(w2s),
        ],
        out_specs=pl.BlockSpec((DIM, TB),
                               lambda i: (0, jnp.maximum(i - 1, 0))),
        out_shape=jax.ShapeDtypeStruct((DIM, HP * WP), jnp.bfloat16),
        scratch_shapes=[pltpu.VMEM((DIM, TB), jnp.float32)],
    )(x2, x2, x2, k, cb, lng, lnb, gate_w, w1s, w2s)

    yc = y.reshape(DIM, HP, WP)[:, 3:3 + H, 3:3 + W]
    return input + ls[None] * yc[None]
